# baseline (speedup 1.0000x reference)
import functools

import jax
import jax.numpy as jnp
from jax import lax
from jax.experimental import pallas as pl
from jax.experimental.pallas import tpu as pltpu
from jax.experimental.pallas import tpu_sc as plsc

_NC = 2
_NS = 16
_NW = _NC * _NS
_L = 16
_CHUNK = 128
_NBUF = 4


def _sc_body(n, d, x_hbm, r_hbm, m_hbm, t_hbm, out_hbm,
             xbuf0, xbuf1, xbuf2, xbuf3, ibuf0, ibuf1,
             rbuf, mbuf, t_sh, sems):
    xbufs = (xbuf0, xbuf1, xbuf2, xbuf3)
    ibufs = (ibuf0, ibuf1)
    rows_per_w = n // _NW
    nchunk = rows_per_w // _CHUNK
    sid = lax.axis_index("s")
    wid = sid * _NC + lax.axis_index("c")
    base_row = wid * rows_per_w
    lanes = lax.iota(jnp.int32, _L)

    @pl.when(sid == 0)
    def _():
        pltpu.sync_copy(t_hbm, t_sh)
    plsc.subcore_barrier()

    def start_in(g):
        slot = g % _NBUF
        row0 = base_row + g * _CHUNK
        h1 = pltpu.async_copy(x_hbm.at[pl.ds(row0, _CHUNK), :],
                              xbufs[slot], sems[slot])
        h2 = pltpu.async_copy(r_hbm.at[pl.ds(row0, _CHUNK)],
                              rbuf.at[slot], sems[slot])
        mrow0 = wid * (rows_per_w // 4) + g * (_CHUNK // 4)
        h3 = pltpu.async_copy(m_hbm.at[pl.ds(mrow0, _CHUNK // 4)],
                              mbuf.at[slot], sems[slot])
        return (h1, h2, h3)

    def start_out(g):
        slot = g % _NBUF
        row0 = base_row + g * _CHUNK
        return pltpu.async_copy(xbufs[slot],
                                out_hbm.at[pl.ds(row0, _CHUNK), :],
                                sems[_NBUF + slot])

    def start_gather_add(g):
        slot = g % _NBUF
        ib = ibufs[g % 2]
        for g16 in range(_CHUNK // _L):
            r0 = g16 * _L
            rv = rbuf[slot, pl.ds(r0, _L)]
            mw = plsc.load_gather(
                mbuf, [jnp.full((_L,), slot, jnp.int32),
                       jnp.full((_L,), r0 // 4, jnp.int32) + (lanes >> 2)])
            mbit = (mw >> ((lanes & 3) * 8)) & 0xFF
            ib[pl.ds(r0, _L)] = jnp.where(
                mbit != 0, jnp.where(rv >= 94, 1, 0), -1)
        return pltpu.async_copy(
            t_sh.at[plsc.Indices(ib, ignored_value=-1)], xbufs[slot],
            sems[2 * _NBUF + g % 2], add=True)

    in_h, out_h, ga_h = {}, {}, {}

    def try_start_in(g):
        if g < nchunk:
            if g >= _NBUF:
                out_h[g - _NBUF].wait()
            in_h[g] = start_in(g)

    for g in range(3):
        try_start_in(g)
    for h in in_h[0]:
        h.wait()
    ga_h[0] = start_gather_add(0)
    for g in range(nchunk):
        if g + 1 < nchunk:
            for h in in_h[g + 1]:
                h.wait()
            ga_h[g + 1] = start_gather_add(g + 1)
        ga_h[g].wait()
        out_h[g] = start_out(g)
        try_start_in(g + 3)
    for g in range(max(0, nchunk - _NBUF), nchunk):
        out_h[g].wait()


def kernel(x, resids, mask, table):
    n, d = x.shape
    r32 = resids.astype(jnp.int32)
    mwords = jax.lax.bitcast_convert_type(
        mask.reshape(n // 4, 4).view(jnp.int8), jnp.int32)

    mesh = plsc.VectorSubcoreMesh(core_axis_name="c", subcore_axis_name="s",
                                  num_cores=_NC, num_subcores=_NS)
    sc = functools.partial(
        pl.kernel,
        out_type=jax.ShapeDtypeStruct((n, d), jnp.float32),
        mesh=mesh,
        compiler_params=pltpu.CompilerParams(needs_layout_passes=False),
        scratch_types=[
            pltpu.VMEM((_CHUNK, d), jnp.float32),
            pltpu.VMEM((_CHUNK, d), jnp.float32),
            pltpu.VMEM((_CHUNK, d), jnp.float32),
            pltpu.VMEM((_CHUNK, d), jnp.float32),
            pltpu.VMEM((_CHUNK,), jnp.int32),
            pltpu.VMEM((_CHUNK,), jnp.int32),
            pltpu.VMEM((_NBUF, _CHUNK), jnp.int32),
            pltpu.VMEM((_NBUF, _CHUNK // 4), jnp.int32),
            pltpu.VMEM_SHARED((2, d), jnp.float32),
            [pltpu.SemaphoreType.DMA] * (2 * _NBUF + 2),
        ],
    )(functools.partial(_sc_body, n, d))
    return sc(x, r32, mwords, table)


# speedup vs baseline: 1.1328x; 1.1328x over previous
import functools

import jax
import jax.numpy as jnp
from jax import lax
from jax.experimental import pallas as pl
from jax.experimental.pallas import tpu as pltpu
from jax.experimental.pallas import tpu_sc as plsc

_NC = 2
_NS = 16
_NW = _NC * _NS
_L = 16
_CHUNK = 128
_NBUF = 4


def _sc_body(n, d, x_hbm, r_hbm, m_hbm, t_hbm, out_hbm,
             xbuf0, xbuf1, xbuf2, xbuf3, ibuf0, ibuf1,
             rbuf, mbuf, t_sh, sems):
    xbufs = (xbuf0, xbuf1, xbuf2, xbuf3)
    ibufs = (ibuf0, ibuf1)
    rows_per_w = n // _NW
    nchunk = rows_per_w // _CHUNK
    sid = lax.axis_index("s")
    wid = sid * _NC + lax.axis_index("c")
    base_row = wid * rows_per_w

    @pl.when(sid == 0)
    def _():
        pltpu.make_async_copy(t_hbm, t_sh, sems[2 * _NBUF]).start()

    def start_in(g):
        slot = g % _NBUF
        row0 = base_row + g * _CHUNK
        h1 = pltpu.async_copy(x_hbm.at[pl.ds(row0, _CHUNK), :],
                              xbufs[slot], sems[slot])
        h2 = pltpu.async_copy(r_hbm.at[pl.ds(row0, _CHUNK)],
                              rbuf.at[slot], sems[slot])
        h3 = pltpu.async_copy(m_hbm.at[pl.ds(row0, _CHUNK)],
                              mbuf.at[slot], sems[slot])
        return (h1, h2, h3)

    def start_out(g):
        slot = g % _NBUF
        row0 = base_row + g * _CHUNK
        return pltpu.async_copy(xbufs[slot],
                                out_hbm.at[pl.ds(row0, _CHUNK), :],
                                sems[_NBUF + slot])

    def start_gather_add(g):
        slot = g % _NBUF
        ib = ibufs[g % 2]
        for g16 in range(_CHUNK // _L):
            r0 = g16 * _L
            rv = rbuf[slot, pl.ds(r0, _L)]
            mv = mbuf[slot, pl.ds(r0, _L)]
            ib[pl.ds(r0, _L)] = jnp.where(
                mv != 0, jnp.where(rv >= 94, 1, 0), -1)
        return pltpu.async_copy(
            t_sh.at[plsc.Indices(ib, ignored_value=-1)], xbufs[slot],
            sems[2 * _NBUF + 1 + g % 2], add=True)

    in_h, out_h, ga_h = {}, {}, {}

    def try_start_in(g):
        if g < nchunk:
            if g >= _NBUF:
                out_h[g - _NBUF].wait()
            in_h[g] = start_in(g)

    for g in range(3):
        try_start_in(g)
    for h in in_h[0]:
        h.wait()
    @pl.when(sid == 0)
    def _():
        pltpu.make_async_copy(t_hbm, t_sh, sems[2 * _NBUF]).wait()
    plsc.subcore_barrier()
    ga_h[0] = start_gather_add(0)
    for g in range(nchunk):
        if g + 1 < nchunk:
            for h in in_h[g + 1]:
                h.wait()
            ga_h[g + 1] = start_gather_add(g + 1)
        ga_h[g].wait()
        out_h[g] = start_out(g)
        try_start_in(g + 3)
    for g in range(max(0, nchunk - _NBUF), nchunk):
        out_h[g].wait()


def kernel(x, resids, mask, table):
    n, d = x.shape
    r32 = resids.astype(jnp.int32)
    m32 = mask.astype(jnp.int32)

    mesh = plsc.VectorSubcoreMesh(core_axis_name="c", subcore_axis_name="s",
                                  num_cores=_NC, num_subcores=_NS)
    sc = functools.partial(
        pl.kernel,
        out_type=jax.ShapeDtypeStruct((n, d), jnp.float32),
        mesh=mesh,
        compiler_params=pltpu.CompilerParams(needs_layout_passes=False),
        scratch_types=[
            pltpu.VMEM((_CHUNK, d), jnp.float32),
            pltpu.VMEM((_CHUNK, d), jnp.float32),
            pltpu.VMEM((_CHUNK, d), jnp.float32),
            pltpu.VMEM((_CHUNK, d), jnp.float32),
            pltpu.VMEM((_CHUNK,), jnp.int32),
            pltpu.VMEM((_CHUNK,), jnp.int32),
            pltpu.VMEM((_NBUF, _CHUNK), jnp.int32),
            pltpu.VMEM((_NBUF, _CHUNK), jnp.int32),
            pltpu.VMEM_SHARED((2, d), jnp.float32),
            [pltpu.SemaphoreType.DMA] * (2 * _NBUF + 3),
        ],
    )(functools.partial(_sc_body, n, d))
    return sc(x, r32, m32, table)


# speedup vs baseline: 1.1577x; 1.0220x over previous
import functools

import jax
import jax.numpy as jnp
from jax import lax
from jax.experimental import pallas as pl
from jax.experimental.pallas import tpu as pltpu
from jax.experimental.pallas import tpu_sc as plsc

_NC = 2
_NS = 16
_NW = _NC * _NS
_L = 16
_CHUNK = 128
_NBUF = 6


def _sc_body(n, d, x_hbm, r_hbm, m_hbm, t_hbm, out_hbm,
             xbuf0, xbuf1, xbuf2, xbuf3, xbuf4, xbuf5, ibuf0, ibuf1,
             rbuf, mbuf, t_sh, sems):
    xbufs = (xbuf0, xbuf1, xbuf2, xbuf3, xbuf4, xbuf5)
    ibufs = (ibuf0, ibuf1)
    rows_per_w = n // _NW
    nchunk = rows_per_w // _CHUNK
    sid = lax.axis_index("s")
    wid = sid * _NC + lax.axis_index("c")
    base_row = wid * rows_per_w

    @pl.when(sid == 0)
    def _():
        pltpu.make_async_copy(t_hbm, t_sh, sems[2 * _NBUF]).start()

    def start_in(g):
        slot = g % _NBUF
        row0 = base_row + g * _CHUNK
        h1 = pltpu.async_copy(x_hbm.at[pl.ds(row0, _CHUNK), :],
                              xbufs[slot], sems[slot])
        h2 = pltpu.async_copy(r_hbm.at[pl.ds(row0, _CHUNK)],
                              rbuf.at[slot], sems[slot])
        h3 = pltpu.async_copy(m_hbm.at[pl.ds(row0, _CHUNK)],
                              mbuf.at[slot], sems[slot])
        return (h1, h2, h3)

    def start_out(g):
        slot = g % _NBUF
        row0 = base_row + g * _CHUNK
        return pltpu.async_copy(xbufs[slot],
                                out_hbm.at[pl.ds(row0, _CHUNK), :],
                                sems[_NBUF + slot])

    def start_gather_add(g):
        slot = g % _NBUF
        ib = ibufs[g % 2]
        for g16 in range(_CHUNK // _L):
            r0 = g16 * _L
            rv = rbuf[slot, pl.ds(r0, _L)]
            mv = mbuf[slot, pl.ds(r0, _L)]
            ib[pl.ds(r0, _L)] = jnp.where(
                mv != 0, jnp.where(rv >= 94, 1, 0), -1)
        return pltpu.async_copy(
            t_sh.at[plsc.Indices(ib, ignored_value=-1)], xbufs[slot],
            sems[2 * _NBUF + 1 + g % 2], add=True)

    in_h, out_h, ga_h = {}, {}, {}

    def try_start_in(g):
        if g < nchunk:
            if g >= _NBUF:
                out_h[g - _NBUF].wait()
            in_h[g] = start_in(g)

    for g in range(5):
        try_start_in(g)
    for h in in_h[0]:
        h.wait()
    @pl.when(sid == 0)
    def _():
        pltpu.make_async_copy(t_hbm, t_sh, sems[2 * _NBUF]).wait()
    plsc.subcore_barrier()
    ga_h[0] = start_gather_add(0)
    for g in range(nchunk):
        if g + 1 < nchunk:
            for h in in_h[g + 1]:
                h.wait()
            ga_h[g + 1] = start_gather_add(g + 1)
        ga_h[g].wait()
        out_h[g] = start_out(g)
        try_start_in(g + 5)
    for g in range(max(0, nchunk - _NBUF), nchunk):
        out_h[g].wait()


def kernel(x, resids, mask, table):
    n, d = x.shape
    r32 = resids.astype(jnp.int32)
    m32 = mask.astype(jnp.int32)

    mesh = plsc.VectorSubcoreMesh(core_axis_name="c", subcore_axis_name="s",
                                  num_cores=_NC, num_subcores=_NS)
    sc = functools.partial(
        pl.kernel,
        out_type=jax.ShapeDtypeStruct((n, d), jnp.float32),
        mesh=mesh,
        compiler_params=pltpu.CompilerParams(needs_layout_passes=False),
        scratch_types=[
            pltpu.VMEM((_CHUNK, d), jnp.float32),
            pltpu.VMEM((_CHUNK, d), jnp.float32),
            pltpu.VMEM((_CHUNK, d), jnp.float32),
            pltpu.VMEM((_CHUNK, d), jnp.float32),
            pltpu.VMEM((_CHUNK, d), jnp.float32),
            pltpu.VMEM((_CHUNK, d), jnp.float32),
            pltpu.VMEM((_CHUNK,), jnp.int32),
            pltpu.VMEM((_CHUNK,), jnp.int32),
            pltpu.VMEM((_NBUF, _CHUNK), jnp.int32),
            pltpu.VMEM((_NBUF, _CHUNK), jnp.int32),
            pltpu.VMEM_SHARED((2, d), jnp.float32),
            [pltpu.SemaphoreType.DMA] * (2 * _NBUF + 3),
        ],
    )(functools.partial(_sc_body, n, d))
    return sc(x, r32, m32, table)


# speedup vs baseline: 1.2170x; 1.0512x over previous
import functools

import jax
import jax.numpy as jnp
from jax import lax
from jax.experimental import pallas as pl
from jax.experimental.pallas import tpu as pltpu
from jax.experimental.pallas import tpu_sc as plsc

_NC = 2
_NS = 16
_NW = _NC * _NS
_L = 16
_CHUNK = 128
_NBUF = 7


def _sc_body(n, d, x_hbm, r_hbm, m_hbm, t_hbm, out_hbm,
             xbuf0, xbuf1, xbuf2, xbuf3, xbuf4, xbuf5, xbuf6, ibuf0, ibuf1,
             rbuf, mbuf, t_sh, sems):
    xbufs = (xbuf0, xbuf1, xbuf2, xbuf3, xbuf4, xbuf5, xbuf6)
    ibufs = (ibuf0, ibuf1)
    rows_per_w = n // _NW
    nchunk = rows_per_w // _CHUNK
    sid = lax.axis_index("s")
    wid = sid * _NC + lax.axis_index("c")
    base_row = wid * rows_per_w

    @pl.when(sid == 0)
    def _():
        pltpu.make_async_copy(t_hbm, t_sh, sems[2 * _NBUF]).start()

    def start_in(g):
        slot = g % _NBUF
        row0 = base_row + g * _CHUNK
        h1 = pltpu.async_copy(x_hbm.at[pl.ds(row0, _CHUNK), :],
                              xbufs[slot], sems[slot])
        h2 = pltpu.async_copy(r_hbm.at[pl.ds(row0, _CHUNK)],
                              rbuf.at[slot], sems[slot])
        h3 = pltpu.async_copy(m_hbm.at[pl.ds(row0, _CHUNK)],
                              mbuf.at[slot], sems[slot])
        return (h1, h2, h3)

    def start_out(g):
        slot = g % _NBUF
        row0 = base_row + g * _CHUNK
        return pltpu.async_copy(xbufs[slot],
                                out_hbm.at[pl.ds(row0, _CHUNK), :],
                                sems[_NBUF + slot])

    def start_gather_add(g):
        slot = g % _NBUF
        ib = ibufs[g % 2]
        for g16 in range(_CHUNK // _L):
            r0 = g16 * _L
            rv = rbuf[slot, pl.ds(r0, _L)]
            mv = mbuf[slot, pl.ds(r0, _L)]
            ib[pl.ds(r0, _L)] = jnp.where(
                mv != 0, jnp.where(rv >= 94, 1, 0), -1)
        return pltpu.async_copy(
            t_sh.at[plsc.Indices(ib, ignored_value=-1)], xbufs[slot],
            sems[2 * _NBUF + 1 + g % 2], add=True)

    in_h, out_h, ga_h = {}, {}, {}

    def try_start_in(g):
        if g < nchunk:
            if g >= _NBUF:
                out_h[g - _NBUF].wait()
            in_h[g] = start_in(g)

    for g in range(6):
        try_start_in(g)
    for h in in_h[0]:
        h.wait()
    @pl.when(sid == 0)
    def _():
        pltpu.make_async_copy(t_hbm, t_sh, sems[2 * _NBUF]).wait()
    plsc.subcore_barrier()
    ga_h[0] = start_gather_add(0)
    for g in range(nchunk):
        if g + 1 < nchunk:
            for h in in_h[g + 1]:
                h.wait()
            ga_h[g + 1] = start_gather_add(g + 1)
        ga_h[g].wait()
        out_h[g] = start_out(g)
        try_start_in(g + 6)
    for g in range(max(0, nchunk - _NBUF), nchunk):
        out_h[g].wait()


def kernel(x, resids, mask, table):
    n, d = x.shape
    r32 = resids.astype(jnp.int32)
    m32 = mask.astype(jnp.int32)

    mesh = plsc.VectorSubcoreMesh(core_axis_name="c", subcore_axis_name="s",
                                  num_cores=_NC, num_subcores=_NS)
    sc = functools.partial(
        pl.kernel,
        out_type=jax.ShapeDtypeStruct((n, d), jnp.float32),
        mesh=mesh,
        compiler_params=pltpu.CompilerParams(needs_layout_passes=False),
        scratch_types=[
            pltpu.VMEM((_CHUNK, d), jnp.float32),
            pltpu.VMEM((_CHUNK, d), jnp.float32),
            pltpu.VMEM((_CHUNK, d), jnp.float32),
            pltpu.VMEM((_CHUNK, d), jnp.float32),
            pltpu.VMEM((_CHUNK, d), jnp.float32),
            pltpu.VMEM((_CHUNK, d), jnp.float32),
            pltpu.VMEM((_CHUNK, d), jnp.float32),
            pltpu.VMEM((_CHUNK,), jnp.int32),
            pltpu.VMEM((_CHUNK,), jnp.int32),
            pltpu.VMEM((_NBUF, _CHUNK), jnp.int32),
            pltpu.VMEM((_NBUF, _CHUNK), jnp.int32),
            pltpu.VMEM_SHARED((2, d), jnp.float32),
            [pltpu.SemaphoreType.DMA] * (2 * _NBUF + 3),
        ],
    )(functools.partial(_sc_body, n, d))
    return sc(x, r32, m32, table)
